# baseline (speedup 1.0000x reference)
import jax
import jax.numpy as jnp
import numpy as np
from jax import lax
from jax.experimental import pallas as pl

_N = 100
_W = 128


def _nms_body(x_ref, sel_ref, cm_ref, nd_ref, box_ref, sc_ref, cls_ref):
    X = x_ref[...]
    sel = sel_ref[...]

    b_iota = lax.broadcasted_iota(jnp.int32, (8, _W), 0)
    i_iota = lax.broadcasted_iota(jnp.int32, (8, _W), 1)
    mask = (sel == b_iota) & (i_iota < _N)

    zpad = jnp.zeros((8, _W - _N), jnp.float32)

    def widen(v):
        return jnp.concatenate([v, zpad], axis=1)

    ch = [X[:, _N:2 * _N, k] for k in range(6)]

    prod = widen(ch[4] * ch[5])
    s_full = jnp.where(mask, prod, jnp.where(i_iota <= _N, 0.0, -1.0))

    boxes = []
    for c in range(4):
        acc = ch[0] * cm_ref[0:1, c:c + 1]
        for k in range(1, 4):
            acc = acc + ch[k] * cm_ref[k:k + 1, c:c + 1]
        boxes.append(jnp.where(mask, widen(acc), 0.0))

    labels = jnp.where(mask, 0, -1).astype(jnp.int32)

    s_k = s_full[:, :, None]
    s_j = s_full[:, None, :]
    km = lax.broadcasted_iota(jnp.int32, (8, _W, _W), 1)
    jm = lax.broadcasted_iota(jnp.int32, (8, _W, _W), 2)
    before = (s_k > s_j) | ((s_k == s_j) & (km < jm))
    rank = jnp.sum(jnp.where(before, 1, 0).astype(jnp.int32), axis=1)

    r_iota = lax.broadcasted_iota(jnp.int32, (8, _W, _W), 1)
    P = rank[:, None, :] == r_iota

    def unsort(v):
        zero = jnp.zeros((), v.dtype)
        return jnp.sum(jnp.where(P, v[:, None, :], zero), axis=2)

    for c in range(4):
        box_ref[:, :, c:c + 1] = unsort(boxes[c])[:, :_N + 1, None]
    sc_ref[...] = unsort(s_full)[:, :_N + 1]
    cls_ref[...] = unsort(labels)[:, :_N + 1]
    nd_ref[...] = jnp.sum(jnp.where(s_full > 0, 1, 0).astype(jnp.int32),
                          axis=1, keepdims=True)


def kernel(x, convert_matrix):
    batch = x.shape[0]

    with jax.ensure_compile_time_eval():
        key = jax.random.key(42)
        sel_b = np.asarray(jnp.sort(jax.random.randint(key, (_N,), 0, batch)))
    sel_np = np.full((batch, _W), batch + 7, np.int32)
    sel_np[:, :_N] = sel_b[None, :]
    sel_pad = jnp.asarray(sel_np)

    out_shapes = (
        jax.ShapeDtypeStruct((batch, 1), jnp.int32),
        jax.ShapeDtypeStruct((batch, _N + 1, 4), jnp.float32),
        jax.ShapeDtypeStruct((batch, _N + 1), jnp.float32),
        jax.ShapeDtypeStruct((batch, _N + 1), jnp.int32),
    )
    num_det, det_boxes, det_scores, det_classes = pl.pallas_call(
        _nms_body,
        out_shape=out_shapes,
        grid=(1,),
        in_specs=[
            pl.BlockSpec((batch, 2 * _N, 85), lambda i: (0, 0, 0)),
            pl.BlockSpec((batch, _W), lambda i: (0, 0)),
            pl.BlockSpec((4, 4), lambda i: (0, 0)),
        ],
        out_specs=(
            pl.BlockSpec((batch, 1), lambda i: (0, 0)),
            pl.BlockSpec((batch, _N + 1, 4), lambda i: (0, 0, 0)),
            pl.BlockSpec((batch, _N + 1), lambda i: (0, 0)),
            pl.BlockSpec((batch, _N + 1), lambda i: (0, 0)),
        ),
    )(x, sel_pad, convert_matrix.astype(jnp.float32))
    return (num_det, det_boxes, det_scores, det_classes)


# speedup vs baseline: 9.8658x; 9.8658x over previous
import jax
import jax.numpy as jnp
import numpy as np
from jax import lax
from jax.experimental import pallas as pl

_N = 100
_W = 128


def _nms_body(x_ref, sel_ref, cm_ref, nd_ref, box_ref, sc_ref, cls_ref):
    X = x_ref[...]
    sel = sel_ref[...]

    b_iota = lax.broadcasted_iota(jnp.int32, (8, _W), 0)
    i_iota = lax.broadcasted_iota(jnp.int32, (8, _W), 1)
    mask = (sel == b_iota) & (i_iota < _N)

    prod = X[:, 4, :] * X[:, 5, :]
    s_full = jnp.where(mask, prod, jnp.where(i_iota <= _N, 0.0, -1.0))

    boxes = []
    for c in range(4):
        acc = X[:, 0, :] * cm_ref[0:1, c:c + 1]
        for k in range(1, 4):
            acc = acc + X[:, k, :] * cm_ref[k:k + 1, c:c + 1]
        boxes.append(jnp.where(mask, acc, 0.0))

    labels = jnp.where(mask, 0, -1).astype(jnp.int32)

    s_k = s_full[:, :, None]
    s_j = s_full[:, None, :]
    km = lax.broadcasted_iota(jnp.int32, (8, _W, _W), 1)
    jm = lax.broadcasted_iota(jnp.int32, (8, _W, _W), 2)
    before = (s_k > s_j) | ((s_k == s_j) & (km < jm))
    rank = jnp.sum(jnp.where(before, 1, 0).astype(jnp.int32), axis=1)

    r_iota = lax.broadcasted_iota(jnp.int32, (8, _W, _W), 1)
    P = rank[:, None, :] == r_iota

    def unsort(v):
        zero = jnp.zeros((), v.dtype)
        return jnp.sum(jnp.where(P, v[:, None, :], zero), axis=2)

    for c in range(4):
        box_ref[:, :, c:c + 1] = unsort(boxes[c])[:, :_N + 1, None]
    sc_ref[...] = unsort(s_full)[:, :_N + 1]
    cls_ref[...] = unsort(labels)[:, :_N + 1]
    nd_ref[...] = jnp.sum(jnp.where(s_full > 0, 1, 0).astype(jnp.int32),
                          axis=1, keepdims=True)


def kernel(x, convert_matrix):
    batch = x.shape[0]

    with jax.ensure_compile_time_eval():
        key = jax.random.key(42)
        sel_b = np.asarray(jnp.sort(jax.random.randint(key, (_N,), 0, batch)))
    sel_np = np.full((batch, _W), batch + 7, np.int32)
    sel_np[:, :_N] = sel_b[None, :]
    sel_pad = jnp.asarray(sel_np)

    xs = lax.slice(x, (0, _N, 0), (batch, 2 * _N, 8))
    xsT = jnp.transpose(xs, (0, 2, 1))
    X8 = jnp.zeros((batch, 8, _W), jnp.float32).at[:, :, :_N].set(xsT)

    out_shapes = (
        jax.ShapeDtypeStruct((batch, 1), jnp.int32),
        jax.ShapeDtypeStruct((batch, _N + 1, 4), jnp.float32),
        jax.ShapeDtypeStruct((batch, _N + 1), jnp.float32),
        jax.ShapeDtypeStruct((batch, _N + 1), jnp.int32),
    )
    num_det, det_boxes, det_scores, det_classes = pl.pallas_call(
        _nms_body,
        out_shape=out_shapes,
    )(X8, sel_pad, convert_matrix.astype(jnp.float32))
    return (num_det, det_boxes, det_scores, det_classes)
